# Initial kernel scaffold; baseline (speedup 1.0000x reference)
import jax
import jax.numpy as jnp
from jax.experimental import pallas as pl


def kernel(x, edge_index, weight, batch, W1, b1, W2, b2, g1, be1, g2, be2, Wf, bf):
    raise NotImplementedError("write your pallas kernel here")


# speedup vs baseline: 9.4328x; 9.4328x over previous
import functools

import jax
import jax.numpy as jnp
from jax import lax
from jax.experimental import pallas as pl
from jax.experimental.pallas import tpu as pltpu
from jax.experimental.pallas import tpu_sc as plsc

NC = 2
NS = 16
LL = 16
NW = NC * NS


def _mesh():
    return plsc.VectorSubcoreMesh(
        core_axis_name="c", subcore_axis_name="s",
        num_cores=NC, num_subcores=NS)


def _sc_deg(dst, w, n):
    e = dst.shape[0]
    ept = e // NW

    nr = n // LL

    @functools.partial(
        pl.kernel,
        out_type=jax.ShapeDtypeStruct((NW, nr, LL), jnp.float32),
        mesh=_mesh(),
        compiler_params=pltpu.CompilerParams(needs_layout_passes=False),
        scratch_types=[
            pltpu.VMEM((ept,), jnp.int32),
            pltpu.VMEM((ept,), jnp.float32),
            pltpu.VMEM((nr, LL), jnp.float32),
        ],
    )
    def k(dst_hbm, w_hbm, out_hbm, idx_v, w_v, deg_v):
        wid = lax.axis_index("s") * NC + lax.axis_index("c")
        base = wid * ept
        zero16 = jnp.zeros((LL,), jnp.float32)

        def zb(i, _):
            deg_v[i, pl.ds(0, LL)] = zero16
            return 0
        lax.fori_loop(0, nr, zb, 0)

        pltpu.sync_copy(dst_hbm.at[pl.ds(base, ept)], idx_v)
        pltpu.sync_copy(w_hbm.at[pl.ds(base, ept)], w_v)

        lane = lax.iota(jnp.int32, LL)

        def body(j, _):
            i16 = idx_v[pl.ds(j * LL, LL)]
            w16 = w_v[pl.ds(j * LL, LL)]
            row = i16 >> 4
            col = i16 & (LL - 1)
            for l in range(LL):
                plsc.addupdate_scatter(
                    deg_v, [row, col], w16, mask=(lane == l))
            return 0
        lax.fori_loop(0, ept // LL, body, 0)
        pltpu.sync_copy(deg_v, out_hbm.at[wid])

    return k(dst, w)


def _sc_agg(hs, src, dst, w, n, f):
    e = src.shape[0]
    ept = e // NW
    ch = 80
    nch = ept // ch
    npad = ((n + 127) // 128) * 128
    rpt = npad // NS

    @functools.partial(
        pl.kernel,
        out_type=jax.ShapeDtypeStruct((NC, npad, f), jnp.float32),
        mesh=_mesh(),
        compiler_params=pltpu.CompilerParams(needs_layout_passes=False),
        scratch_types=[
            pltpu.VMEM_SHARED((npad, f), jnp.float32),
            pltpu.VMEM((ch,), jnp.int32),
            pltpu.VMEM((ch,), jnp.int32),
            pltpu.VMEM((ch,), jnp.float32),
            pltpu.VMEM((ch, f), jnp.float32),
            pltpu.SemaphoreType.DMA,
        ],
    )
    def k(hs_hbm, src_hbm, dst_hbm, w_hbm, out_hbm,
          acc, si_v, di_v, w_v, rows_v, sem):
        core = lax.axis_index("c")
        sub = lax.axis_index("s")
        wid = sub * NC + core
        base_e = wid * ept
        fch = f // LL
        zero16 = jnp.zeros((LL,), jnp.float32)

        def zb(t, _):
            rows_v[t // fch, pl.ds((t % fch) * LL, LL)] = zero16
            return 0
        lax.fori_loop(0, ch * fch, zb, 0)
        nfull = rpt // ch
        for i in range(nfull):
            pltpu.sync_copy(rows_v, acc.at[pl.ds(sub * rpt + i * ch, ch)])
        rem = rpt - nfull * ch
        if rem:
            pltpu.sync_copy(rows_v.at[pl.ds(0, rem)],
                            acc.at[pl.ds(sub * rpt + nfull * ch, rem)])
        plsc.subcore_barrier()

        def chunk(c, _):
            off = base_e + c * ch
            pltpu.sync_copy(src_hbm.at[pl.ds(off, ch)], si_v)
            pltpu.sync_copy(dst_hbm.at[pl.ds(off, ch)], di_v)
            pltpu.sync_copy(w_hbm.at[pl.ds(off, ch)], w_v)
            pltpu.async_copy(hs_hbm.at[si_v], rows_v, sem).wait()

            def srow(j, _):
                wj = plsc.load_gather(w_v, [jnp.full((LL,), j, jnp.int32)])
                for kk in range(fch):
                    sl = pl.ds(kk * LL, LL)
                    rows_v[j, sl] = rows_v[j, sl] * wj
                return 0
            lax.fori_loop(0, ch, srow, 0)
            pltpu.sync_copy(rows_v, acc.at[di_v], add=True)
            return 0
        lax.fori_loop(0, nch, chunk, 0)
        plsc.subcore_barrier()

        pltpu.sync_copy(acc.at[pl.ds(sub * rpt, rpt)],
                        out_hbm.at[core, pl.ds(sub * rpt, rpt)])

    return k(hs, src, dst, w)


def _sc_pool(h2, batch, n, f, g):
    ngroups = n // LL

    @functools.partial(
        pl.kernel,
        out_type=(jax.ShapeDtypeStruct((NW, g, f), jnp.float32),
                  jax.ShapeDtypeStruct((NW, g, f), jnp.float32)),
        mesh=_mesh(),
        compiler_params=pltpu.CompilerParams(needs_layout_passes=False),
        scratch_types=[
            pltpu.VMEM((LL,), jnp.int32),
            pltpu.VMEM((LL, f), jnp.float32),
            pltpu.VMEM((g, f), jnp.float32),
            pltpu.VMEM((g, f), jnp.float32),
        ],
    )
    def k(h2_hbm, batch_hbm, osum, omax, bseg_v, grows_v, psum_v, pmax_v):
        wid = lax.axis_index("s") * NC + lax.axis_index("c")
        fch = f // LL
        zero16 = jnp.zeros((LL,), jnp.float32)
        ninf16 = jnp.full((LL,), -jnp.inf, jnp.float32)

        def zb(t, _):
            psum_v[t // fch, pl.ds((t % fch) * LL, LL)] = zero16
            pmax_v[t // fch, pl.ds((t % fch) * LL, LL)] = ninf16
            return 0
        lax.fori_loop(0, g * fch, zb, 0)

        g0 = (ngroups * wid) // NW
        g1 = (ngroups * (wid + 1)) // NW

        def flush(seg, sums, maxs):
            for kk in range(fch):
                psum_v[seg, pl.ds(kk * LL, LL)] = sums[kk]
                pmax_v[seg, pl.ds(kk * LL, LL)] = maxs[kk]

        def gb(i, carry):
            cur_seg = carry[0]
            sums = list(carry[1:1 + fch])
            maxs = list(carry[1 + fch:])
            r0 = (g0 + i) * LL
            pltpu.sync_copy(batch_hbm.at[pl.ds(r0, LL)], bseg_v)
            pltpu.sync_copy(h2_hbm.at[pl.ds(r0, LL)], grows_v)
            bvec = bseg_v[...]
            for j in range(LL):
                s = bvec[j]
                new_run = s != cur_seg

                @pl.when(new_run & (cur_seg >= 0))
                def _(cs=cur_seg, sv=tuple(sums), mv=tuple(maxs)):
                    flush(cs, sv, mv)
                for kk in range(fch):
                    row = grows_v[j, pl.ds(kk * LL, LL)]
                    sums[kk] = jnp.where(new_run, row, sums[kk] + row)
                    maxs[kk] = jnp.where(new_run, row,
                                         jnp.maximum(maxs[kk], row))
                cur_seg = s
            return (cur_seg, *sums, *maxs)

        init = (jnp.int32(-1),) + (zero16,) * fch + (ninf16,) * fch
        fin = lax.fori_loop(0, g1 - g0, gb, init)
        flush(fin[0], fin[1:1 + fch], fin[1 + fch:])

        pltpu.sync_copy(psum_v, osum.at[wid])
        pltpu.sync_copy(pmax_v, omax.at[wid])

    return k(h2, batch)


def _tc_prep(x, degp_t, W1, n, d, h):
    nb = 10
    br = n // nb

    def body(x_ref, dp_ref, w_ref, hs_ref, dis_ref):
        deg = jnp.sum(dp_ref[...], axis=1, keepdims=True) + 1.0
        dis = jnp.where(deg > 0,
                        lax.rsqrt(jnp.maximum(deg, 1e-12)),
                        0.0)
        hm = jnp.dot(x_ref[...], w_ref[...],
                     preferred_element_type=jnp.float32)
        hs_ref[...] = hm * dis
        dis_ref[...] = dis

    return pl.pallas_call(
        body,
        grid=(nb,),
        in_specs=[
            pl.BlockSpec((br, d), lambda i: (i, 0)),
            pl.BlockSpec((br, NW), lambda i: (i, 0)),
            pl.BlockSpec((d, h), lambda i: (0, 0)),
        ],
        out_specs=[
            pl.BlockSpec((br, h), lambda i: (i, 0)),
            pl.BlockSpec((br, 1), lambda i: (i, 0)),
        ],
        out_shape=[
            jax.ShapeDtypeStruct((n, h), jnp.float32),
            jax.ShapeDtypeStruct((n, 1), jnp.float32),
        ],
    )(x, degp_t, W1)


def _tc_mid(part, hs, dis, n, h):
    nb = 10
    br = n // nb

    def body(p_ref, hs_ref, dis_ref, agg_ref, mom_ref):
        i = pl.program_id(0)
        a = (p_ref[0] + p_ref[1] + hs_ref[...]) * dis_ref[...]
        agg_ref[...] = a

        @pl.when(i == 0)
        def _():
            mom_ref[...] = jnp.zeros((2, h), jnp.float32)
        mom_ref[0:1, :] += jnp.sum(a, axis=0, keepdims=True)
        mom_ref[1:2, :] += jnp.sum(a * a, axis=0, keepdims=True)

    return pl.pallas_call(
        body,
        grid=(nb,),
        in_specs=[
            pl.BlockSpec((NC, br, h), lambda i: (0, i, 0)),
            pl.BlockSpec((br, h), lambda i: (i, 0)),
            pl.BlockSpec((br, 1), lambda i: (i, 0)),
        ],
        out_specs=[
            pl.BlockSpec((br, h), lambda i: (i, 0)),
            pl.BlockSpec((2, h), lambda i: (0, 0)),
        ],
        out_shape=[
            jax.ShapeDtypeStruct((n, h), jnp.float32),
            jax.ShapeDtypeStruct((2, h), jnp.float32),
        ],
    )(part, hs, dis)


def _tc_mid2(agg, mom, g1, be1, W2, dis, n, h, h2dim):
    nb = 10
    br = n // nb
    inv_n = 1.0 / n

    def body(a_ref, mom_ref, g_ref, b_ref, w_ref, dis_ref, o_ref):
        mu = mom_ref[0:1, :] * inv_n
        var = mom_ref[1:2, :] * inv_n - mu * mu
        y = (a_ref[...] - mu) * lax.rsqrt(var + 1e-5) * g_ref[...] + b_ref[...]
        y = y * jax.nn.sigmoid(y)
        hm = jnp.dot(y, w_ref[...], preferred_element_type=jnp.float32)
        o_ref[...] = hm * dis_ref[...]

    return pl.pallas_call(
        body,
        grid=(nb,),
        in_specs=[
            pl.BlockSpec((br, h), lambda i: (i, 0)),
            pl.BlockSpec((2, h), lambda i: (0, 0)),
            pl.BlockSpec((1, h), lambda i: (0, 0)),
            pl.BlockSpec((1, h), lambda i: (0, 0)),
            pl.BlockSpec((h, h2dim), lambda i: (0, 0)),
            pl.BlockSpec((br, 1), lambda i: (i, 0)),
        ],
        out_specs=pl.BlockSpec((br, h2dim), lambda i: (i, 0)),
        out_shape=jax.ShapeDtypeStruct((n, h2dim), jnp.float32),
    )(agg, mom, g1, be1, W2, dis)


def _tc_fin1(part, hs2, dis, b2, n, f):
    nb = 10
    br = n // nb

    def body(p_ref, hs_ref, dis_ref, b_ref, o_ref):
        o_ref[...] = ((p_ref[0] + p_ref[1] + hs_ref[...]) * dis_ref[...]
                      + b_ref[...])

    return pl.pallas_call(
        body,
        grid=(nb,),
        in_specs=[
            pl.BlockSpec((NC, br, f), lambda i: (0, i, 0)),
            pl.BlockSpec((br, f), lambda i: (i, 0)),
            pl.BlockSpec((br, 1), lambda i: (i, 0)),
            pl.BlockSpec((1, f), lambda i: (0, 0)),
        ],
        out_specs=pl.BlockSpec((br, f), lambda i: (i, 0)),
        out_shape=jax.ShapeDtypeStruct((n, f), jnp.float32),
    )(part, hs2, dis, b2)


def _tc_fin2(psum, pmax, batch2d, g2, be2, Wf, bf, n, f, g):
    def body(ps_ref, pm_ref, b_ref, g_ref, be_ref, wf_ref, bf_ref, o_ref):
        s = jnp.sum(ps_ref[...], axis=0)[:, :f]
        m = jnp.max(pm_ref[...], axis=0)[:, :f]
        ids = b_ref[...]
        gi = lax.broadcasted_iota(jnp.int32, (g, n), 0)
        cnt = jnp.sum(jnp.where(gi == ids, 1.0, 0.0), axis=1, keepdims=True)
        mean = s / jnp.maximum(cnt, 1.0)
        z = jnp.concatenate([mean, m], axis=1)
        mu = jnp.mean(z, axis=0, keepdims=True)
        var = jnp.mean((z - mu) * (z - mu), axis=0, keepdims=True)
        zn = (z - mu) * lax.rsqrt(var + 1e-5) * g_ref[...] + be_ref[...]
        o_ref[...] = jnp.dot(zn, wf_ref[...],
                             preferred_element_type=jnp.float32) + bf_ref[...]

    return pl.pallas_call(
        body,
        out_shape=jax.ShapeDtypeStruct((g, 1), jnp.float32),
    )(psum, pmax, batch2d, g2, be2, Wf, bf)


def kernel(x, edge_index, weight, batch, W1, b1, W2, b2,
           g1, be1, g2, be2, Wf, bf):
    n, d = x.shape
    h = W1.shape[1]
    h2dim = W2.shape[1]
    g = 100
    src = edge_index[0]
    dst = edge_index[1]

    W2p = jnp.concatenate(
        [W2, jnp.zeros((h, h - h2dim), jnp.float32)], axis=1)
    b2p = jnp.concatenate([b2, jnp.zeros((h - h2dim,), jnp.float32)])

    degp = _sc_deg(dst, weight, n).reshape(NW, n)
    degp_t = jnp.transpose(degp)
    hs, dis = _tc_prep(x, degp_t, W1, n, d, h)
    part1 = _sc_agg(hs, src, dst, weight, n, h)
    agg, mom = _tc_mid(part1, hs, dis, n, h)
    hs2 = _tc_mid2(agg, mom, g1.reshape(1, h), be1.reshape(1, h),
                   W2p, dis, n, h, h)
    part2 = _sc_agg(hs2, src, dst, weight, n, h)
    h2 = _tc_fin1(part2, hs2, dis, b2p.reshape(1, h), n, h)
    psum, pmax = _sc_pool(h2, batch, n, h, g)
    out = _tc_fin2(psum, pmax, batch.reshape(1, n),
                   g2.reshape(1, 2 * h2dim), be2.reshape(1, 2 * h2dim),
                   Wf, bf.reshape(1, 1), n, h2dim, g)
    return out


# speedup vs baseline: 21.5353x; 2.2830x over previous
import functools

import jax
import jax.numpy as jnp
from jax import lax
from jax.experimental import pallas as pl
from jax.experimental.pallas import tpu as pltpu
from jax.experimental.pallas import tpu_sc as plsc

NC = 2
NS = 16
LL = 16
NW = NC * NS


def _mesh():
    return plsc.VectorSubcoreMesh(
        core_axis_name="c", subcore_axis_name="s",
        num_cores=NC, num_subcores=NS)


def _sc_deg(dst, w, n):
    e = dst.shape[0]
    ept = e // NW

    nr = n // LL

    @functools.partial(
        pl.kernel,
        out_type=jax.ShapeDtypeStruct((NW, nr, LL), jnp.float32),
        mesh=_mesh(),
        compiler_params=pltpu.CompilerParams(needs_layout_passes=False),
        scratch_types=[
            pltpu.VMEM((ept,), jnp.int32),
            pltpu.VMEM((ept,), jnp.float32),
            pltpu.VMEM((nr, LL), jnp.float32),
        ],
    )
    def k(dst_hbm, w_hbm, out_hbm, idx_v, w_v, deg_v):
        wid = lax.axis_index("s") * NC + lax.axis_index("c")
        base = wid * ept
        zero16 = jnp.zeros((LL,), jnp.float32)

        def zb(i, _):
            deg_v[i, pl.ds(0, LL)] = zero16
            return 0
        lax.fori_loop(0, nr, zb, 0)

        pltpu.sync_copy(dst_hbm.at[pl.ds(base, ept)], idx_v)
        pltpu.sync_copy(w_hbm.at[pl.ds(base, ept)], w_v)

        lane = lax.iota(jnp.int32, LL)

        def body(j, _):
            i16 = idx_v[pl.ds(j * LL, LL)]
            w16 = w_v[pl.ds(j * LL, LL)]
            row = i16 >> 4
            col = i16 & (LL - 1)
            for l in range(LL):
                plsc.addupdate_scatter(
                    deg_v, [row, col], w16, mask=(lane == l))
            return 0
        lax.fori_loop(0, ept // LL, body, 0)
        pltpu.sync_copy(deg_v, out_hbm.at[wid])

    return k(dst, w)


def _sc_agg(hs, src, dst, w, n, f, fvalid):
    e = src.shape[0]
    ept = e // NW
    ch = 80
    nch = ept // ch
    nquad = (nch - 1) // 4
    assert nch == 4 * nquad + 1
    npad = ((n + 127) // 128) * 128
    rpt = npad // NS


    @functools.partial(
        pl.kernel,
        out_type=jax.ShapeDtypeStruct((NC, npad, f), jnp.float32),
        mesh=_mesh(),
        compiler_params=pltpu.CompilerParams(needs_layout_passes=False),
        scratch_types=[
            pltpu.VMEM_SHARED((npad, f), jnp.float32),
            pltpu.VMEM((4, ch), jnp.int32),
            pltpu.VMEM((4, ch), jnp.int32),
            pltpu.VMEM((4, ch), jnp.float32),
            pltpu.VMEM((ch, f), jnp.float32),
            pltpu.VMEM((ch, f), jnp.float32),
            pltpu.SemaphoreType.DMA,
            pltpu.SemaphoreType.DMA,
            pltpu.SemaphoreType.DMA,
            pltpu.SemaphoreType.DMA,
            pltpu.SemaphoreType.DMA,
            pltpu.SemaphoreType.DMA,
            pltpu.SemaphoreType.DMA,
            pltpu.SemaphoreType.DMA,
        ],
    )
    def k(hs_hbm, src_hbm, dst_hbm, w_hbm, out_hbm,
          acc, sbufs, dbufs, wbufs, rows0, rows1,
          sg0, sg1, ss0, ss1, se0, se1, se2, se3):
        se = (se0, se1, se2, se3)
        core = lax.axis_index("c")
        sub = lax.axis_index("s")
        wid = sub * NC + core
        fch = f // LL
        nsc = fvalid // LL
        zero16 = jnp.zeros((LL,), jnp.float32)
        rows = (rows0, rows1)
        sg = (sg0, sg1)
        ss = (ss0, ss1)

        def zb(t, _):
            rows0[t // fch, pl.ds((t % fch) * LL, LL)] = zero16
            return 0
        lax.fori_loop(0, ch * fch, zb, 0)
        nfull = rpt // ch
        for i in range(nfull):
            pltpu.sync_copy(rows0, acc.at[pl.ds(sub * rpt + i * ch, ch)])
        rem = rpt - nfull * ch
        if rem:
            pltpu.sync_copy(rows0.at[pl.ds(0, rem)],
                            acc.at[pl.ds(sub * rpt + nfull * ch, rem)])
        plsc.subcore_barrier()

        base_e = wid * ept

        def estart(c, t):
            off = base_e + c * ch
            pltpu.async_copy(src_hbm.at[pl.ds(off, ch)], sbufs.at[t], se[t])
            pltpu.async_copy(dst_hbm.at[pl.ds(off, ch)], dbufs.at[t], se[t])
            pltpu.async_copy(w_hbm.at[pl.ds(off, ch)], wbufs.at[t], se[t])

        def ewait(c, t):
            off = base_e + c * ch
            pltpu.make_async_copy(src_hbm.at[pl.ds(off, ch)], sbufs.at[t],
                                  se[t]).wait()
            pltpu.make_async_copy(dst_hbm.at[pl.ds(off, ch)], dbufs.at[t],
                                  se[t]).wait()
            pltpu.make_async_copy(w_hbm.at[pl.ds(off, ch)], wbufs.at[t],
                                  se[t]).wait()

        def gather_start(t, b):
            pltpu.async_copy(hs_hbm.at[sbufs.at[t]], rows[b], sg[b])

        def gather_wait(t, b):
            pltpu.make_async_copy(hs_hbm.at[sbufs.at[t]], rows[b],
                                  sg[b]).wait()

        def scatter_start(t, b):
            pltpu.async_copy(rows[b], acc.at[dbufs.at[t]], ss[b], add=True)

        def scatter_wait(t, b):
            pltpu.make_async_copy(rows[b], acc.at[dbufs.at[t]],
                                  ss[b]).wait()

        def scale(t, b):
            t16 = jnp.full((LL,), t, jnp.int32)

            def srow(j, _):
                wj = plsc.load_gather(
                    wbufs, [t16, jnp.full((LL,), j, jnp.int32)])
                for kk in range(nsc):
                    sl = pl.ds(kk * LL, LL)
                    rows[b][j, sl] = rows[b][j, sl] * wj
                return 0
            lax.fori_loop(0, ch, srow, 0, unroll=2)

        estart(0, 0)
        estart(1, 1)
        estart(2, 2)
        ewait(0, 0)
        gather_start(0, 0)

        def quad(q, _):
            c0 = 4 * q
            for t in range(4):
                c = c0 + t
                b = t % 2
                gather_wait(t, b)

                @pl.when(c >= 1)
                def _(tt=(t - 1) % 4, bb=1 - b):
                    scatter_wait(tt, bb)

                @pl.when(c + 3 < nch)
                def _(cc=c + 3, tt=(t + 3) % 4):
                    estart(cc, tt)

                @pl.when(c + 1 < nch)
                def _(cc=c + 1, tt=(t + 1) % 4, bb=1 - b):
                    ewait(cc, tt)
                    gather_start(tt, bb)
                scale(t, b)
                scatter_start(t, b)
            return 0
        lax.fori_loop(0, nquad, quad, 0)

        gather_wait(0, 0)
        scatter_wait(3, 1)
        scale(0, 0)
        scatter_start(0, 0)
        scatter_wait(0, 0)
        plsc.subcore_barrier()

        pltpu.sync_copy(acc.at[pl.ds(sub * rpt, rpt)],
                        out_hbm.at[core, pl.ds(sub * rpt, rpt)])

    return k(hs, src, dst, w)


def _sc_pool(h2, batch, n, f, g):
    ngroups = n // LL

    @functools.partial(
        pl.kernel,
        out_type=(jax.ShapeDtypeStruct((NW, g, f), jnp.float32),
                  jax.ShapeDtypeStruct((NW, g, f), jnp.float32)),
        mesh=_mesh(),
        compiler_params=pltpu.CompilerParams(needs_layout_passes=False),
        scratch_types=[
            pltpu.VMEM((LL,), jnp.int32),
            pltpu.VMEM((LL, f), jnp.float32),
            pltpu.VMEM((g, f), jnp.float32),
            pltpu.VMEM((g, f), jnp.float32),
        ],
    )
    def k(h2_hbm, batch_hbm, osum, omax, bseg_v, grows_v, psum_v, pmax_v):
        wid = lax.axis_index("s") * NC + lax.axis_index("c")
        fch = f // LL
        zero16 = jnp.zeros((LL,), jnp.float32)
        ninf16 = jnp.full((LL,), -jnp.inf, jnp.float32)

        def zb(t, _):
            psum_v[t // fch, pl.ds((t % fch) * LL, LL)] = zero16
            pmax_v[t // fch, pl.ds((t % fch) * LL, LL)] = ninf16
            return 0
        lax.fori_loop(0, g * fch, zb, 0)

        g0 = (ngroups * wid) // NW
        g1 = (ngroups * (wid + 1)) // NW

        def flush(seg, sums, maxs):
            for kk in range(fch):
                psum_v[seg, pl.ds(kk * LL, LL)] = sums[kk]
                pmax_v[seg, pl.ds(kk * LL, LL)] = maxs[kk]

        def gb(i, carry):
            cur_seg = carry[0]
            sums = list(carry[1:1 + fch])
            maxs = list(carry[1 + fch:])
            r0 = (g0 + i) * LL
            pltpu.sync_copy(batch_hbm.at[pl.ds(r0, LL)], bseg_v)
            pltpu.sync_copy(h2_hbm.at[pl.ds(r0, LL)], grows_v)
            bvec = bseg_v[...]
            for j in range(LL):
                s = bvec[j]
                new_run = s != cur_seg

                @pl.when(new_run & (cur_seg >= 0))
                def _(cs=cur_seg, sv=tuple(sums), mv=tuple(maxs)):
                    flush(cs, sv, mv)
                for kk in range(fch):
                    row = grows_v[j, pl.ds(kk * LL, LL)]
                    sums[kk] = jnp.where(new_run, row, sums[kk] + row)
                    maxs[kk] = jnp.where(new_run, row,
                                         jnp.maximum(maxs[kk], row))
                cur_seg = s
            return (cur_seg, *sums, *maxs)

        init = (jnp.int32(-1),) + (zero16,) * fch + (ninf16,) * fch
        fin = lax.fori_loop(0, g1 - g0, gb, init)
        flush(fin[0], fin[1:1 + fch], fin[1 + fch:])

        pltpu.sync_copy(psum_v, osum.at[wid])
        pltpu.sync_copy(pmax_v, omax.at[wid])

    return k(h2, batch)


def _tc_prep(x, degp_t, W1, n, d, h):
    nb = 10
    br = n // nb

    def body(x_ref, dp_ref, w_ref, hs_ref, dis_ref):
        deg = jnp.sum(dp_ref[...], axis=1, keepdims=True) + 1.0
        dis = jnp.where(deg > 0,
                        lax.rsqrt(jnp.maximum(deg, 1e-12)),
                        0.0)
        hm = jnp.dot(x_ref[...], w_ref[...],
                     preferred_element_type=jnp.float32)
        hs_ref[...] = hm * dis
        dis_ref[...] = dis

    return pl.pallas_call(
        body,
        grid=(nb,),
        in_specs=[
            pl.BlockSpec((br, d), lambda i: (i, 0)),
            pl.BlockSpec((br, NW), lambda i: (i, 0)),
            pl.BlockSpec((d, h), lambda i: (0, 0)),
        ],
        out_specs=[
            pl.BlockSpec((br, h), lambda i: (i, 0)),
            pl.BlockSpec((br, 1), lambda i: (i, 0)),
        ],
        out_shape=[
            jax.ShapeDtypeStruct((n, h), jnp.float32),
            jax.ShapeDtypeStruct((n, 1), jnp.float32),
        ],
    )(x, degp_t, W1)


def _tc_mid(part, hs, dis, n, h):
    nb = 10
    br = n // nb

    def body(p_ref, hs_ref, dis_ref, agg_ref, mom_ref):
        i = pl.program_id(0)
        a = (p_ref[0] + p_ref[1] + hs_ref[...]) * dis_ref[...]
        agg_ref[...] = a

        @pl.when(i == 0)
        def _():
            mom_ref[...] = jnp.zeros((2, h), jnp.float32)
        mom_ref[0:1, :] += jnp.sum(a, axis=0, keepdims=True)
        mom_ref[1:2, :] += jnp.sum(a * a, axis=0, keepdims=True)

    return pl.pallas_call(
        body,
        grid=(nb,),
        in_specs=[
            pl.BlockSpec((NC, br, h), lambda i: (0, i, 0)),
            pl.BlockSpec((br, h), lambda i: (i, 0)),
            pl.BlockSpec((br, 1), lambda i: (i, 0)),
        ],
        out_specs=[
            pl.BlockSpec((br, h), lambda i: (i, 0)),
            pl.BlockSpec((2, h), lambda i: (0, 0)),
        ],
        out_shape=[
            jax.ShapeDtypeStruct((n, h), jnp.float32),
            jax.ShapeDtypeStruct((2, h), jnp.float32),
        ],
    )(part, hs, dis)


def _tc_mid2(agg, mom, g1, be1, W2, dis, n, h, h2dim):
    nb = 10
    br = n // nb
    inv_n = 1.0 / n

    def body(a_ref, mom_ref, g_ref, b_ref, w_ref, dis_ref, o_ref):
        mu = mom_ref[0:1, :] * inv_n
        var = mom_ref[1:2, :] * inv_n - mu * mu
        y = (a_ref[...] - mu) * lax.rsqrt(var + 1e-5) * g_ref[...] + b_ref[...]
        y = y * jax.nn.sigmoid(y)
        hm = jnp.dot(y, w_ref[...], preferred_element_type=jnp.float32)
        o_ref[...] = hm * dis_ref[...]

    return pl.pallas_call(
        body,
        grid=(nb,),
        in_specs=[
            pl.BlockSpec((br, h), lambda i: (i, 0)),
            pl.BlockSpec((2, h), lambda i: (0, 0)),
            pl.BlockSpec((1, h), lambda i: (0, 0)),
            pl.BlockSpec((1, h), lambda i: (0, 0)),
            pl.BlockSpec((h, h2dim), lambda i: (0, 0)),
            pl.BlockSpec((br, 1), lambda i: (i, 0)),
        ],
        out_specs=pl.BlockSpec((br, h2dim), lambda i: (i, 0)),
        out_shape=jax.ShapeDtypeStruct((n, h2dim), jnp.float32),
    )(agg, mom, g1, be1, W2, dis)


def _tc_fin1(part, hs2, dis, b2, n, f):
    nb = 10
    br = n // nb

    def body(p_ref, hs_ref, dis_ref, b_ref, o_ref):
        o_ref[...] = ((p_ref[0] + p_ref[1] + hs_ref[...]) * dis_ref[...]
                      + b_ref[...])

    return pl.pallas_call(
        body,
        grid=(nb,),
        in_specs=[
            pl.BlockSpec((NC, br, f), lambda i: (0, i, 0)),
            pl.BlockSpec((br, f), lambda i: (i, 0)),
            pl.BlockSpec((br, 1), lambda i: (i, 0)),
            pl.BlockSpec((1, f), lambda i: (0, 0)),
        ],
        out_specs=pl.BlockSpec((br, f), lambda i: (i, 0)),
        out_shape=jax.ShapeDtypeStruct((n, f), jnp.float32),
    )(part, hs2, dis, b2)


def _tc_fin2(psum, pmax, batch2d, g2, be2, Wf, bf, n, f, g):
    def body(ps_ref, pm_ref, b_ref, g_ref, be_ref, wf_ref, bf_ref, o_ref):
        s = jnp.sum(ps_ref[...], axis=0)[:, :f]
        m = jnp.max(pm_ref[...], axis=0)[:, :f]
        ids = b_ref[...]
        gi = lax.broadcasted_iota(jnp.int32, (g, n), 0)
        cnt = jnp.sum(jnp.where(gi == ids, 1.0, 0.0), axis=1, keepdims=True)
        mean = s / jnp.maximum(cnt, 1.0)
        z = jnp.concatenate([mean, m], axis=1)
        mu = jnp.mean(z, axis=0, keepdims=True)
        var = jnp.mean((z - mu) * (z - mu), axis=0, keepdims=True)
        zn = (z - mu) * lax.rsqrt(var + 1e-5) * g_ref[...] + be_ref[...]
        o_ref[...] = jnp.dot(zn, wf_ref[...],
                             preferred_element_type=jnp.float32) + bf_ref[...]

    return pl.pallas_call(
        body,
        out_shape=jax.ShapeDtypeStruct((g, 1), jnp.float32),
    )(psum, pmax, batch2d, g2, be2, Wf, bf)


def kernel(x, edge_index, weight, batch, W1, b1, W2, b2,
           g1, be1, g2, be2, Wf, bf):
    n, d = x.shape
    h = W1.shape[1]
    h2dim = W2.shape[1]
    g = 100
    src = edge_index[0]
    dst = edge_index[1]

    W2p = jnp.concatenate(
        [W2, jnp.zeros((h, h - h2dim), jnp.float32)], axis=1)
    b2p = jnp.concatenate([b2, jnp.zeros((h - h2dim,), jnp.float32)])

    degp = _sc_deg(dst, weight, n).reshape(NW, n)
    degp_t = jnp.transpose(degp)
    hs, dis = _tc_prep(x, degp_t, W1, n, d, h)
    part1 = _sc_agg(hs, src, dst, weight, n, h, h)
    agg, mom = _tc_mid(part1, hs, dis, n, h)
    hs2 = _tc_mid2(agg, mom, g1.reshape(1, h), be1.reshape(1, h),
                   W2p, dis, n, h, h)
    part2 = _sc_agg(hs2, src, dst, weight, n, h, h2dim)
    h2 = _tc_fin1(part2, hs2, dis, b2p.reshape(1, h), n, h)
    psum, pmax = _sc_pool(h2, batch, n, h, g)
    out = _tc_fin2(psum, pmax, batch.reshape(1, n),
                   g2.reshape(1, 2 * h2dim), be2.reshape(1, 2 * h2dim),
                   Wf, bf.reshape(1, 1), n, h2dim, g)
    return out
